# Initial kernel scaffold; baseline (speedup 1.0000x reference)
import jax
import jax.numpy as jnp
from jax.experimental import pallas as pl


def kernel(bay, T, W_c, b_c, W_t, b_t, token_table):
    raise NotImplementedError("write your pallas kernel here")


# speedup vs baseline: 1.2795x; 1.2795x over previous
import numpy as np
import jax
import jax.numpy as jnp
from jax import lax
from jax.experimental import pallas as pl

_B, _C, _R = 128, 24, 20
_NPORTS = 16
_M = _NPORTS * (_NPORTS - 1) // 2
_D = 256
_BAYL = _C * (_R + 1)
_L = _BAYL + _M + (_NPORTS - 2) + 1
_NS = _C * _R + _M
_NSP = 608


def _build_maps():
    src = np.full((_L,), -1, np.int64)
    biasrow = np.zeros((_L,), np.int64)
    for c in range(_C):
        for r in range(_R):
            p = c * (_R + 1) + r
            src[p] = c * _R + r
            biasrow[p] = 0
        p = c * (_R + 1) + _R
        biasrow[p] = 2 + 0
    biasrow[_BAYL - 1] = 2 + 1
    pos = _BAYL
    idx = 0
    for row_len in range(_NPORTS - 1, 0, -1):
        for _ in range(row_len):
            src[pos] = _C * _R + idx
            biasrow[pos] = 1
            idx += 1
            pos += 1
        if idx != _M:
            biasrow[pos] = 2 + 2
            pos += 1
    biasrow[pos] = 2 + 3
    pos += 1
    assert pos == _L
    return src, biasrow


_SRC, _BIASROW = _build_maps()

_P_np = np.zeros((_NSP, _L), np.float32)
for _p in range(_L):
    if _SRC[_p] >= 0:
        _P_np[_SRC[_p], _p] = 1.0
_OH_np = np.zeros((_L, 8), np.float32)
_OH_np[np.arange(_L), _BIASROW] = 1.0

_BB = 16


def _embed_kernel(scal_ref, p_ref, oh_ref, small_ref, wc_ref, wt_ref, out_ref):
    sg = jnp.dot(scal_ref[...], p_ref[...],
                 preferred_element_type=jnp.float32,
                 precision=lax.Precision.HIGHEST)
    bias = jnp.dot(oh_ref[...], small_ref[...],
                   preferred_element_type=jnp.float32,
                   precision=lax.Precision.HIGHEST)
    pos = lax.broadcasted_iota(jnp.int32, (_L, 1), 0)
    wsel = jnp.where(pos < _BAYL, wc_ref[...], wt_ref[...])
    out_ref[...] = sg[:, :, None] * wsel[None, :, :] + bias[None, :, :]


def kernel(bay, T, W_c, b_c, W_t, b_t, token_table):
    bay2 = bay.reshape(_B, _C * _R)
    T2 = T.reshape(_B, _M)
    scal = jnp.concatenate(
        [bay2, T2, jnp.zeros((_B, _NSP - _NS), jnp.float32)], axis=1)
    small = jnp.concatenate(
        [b_c[None, :], b_t[None, :], token_table,
         jnp.zeros((2, _D), jnp.float32)], axis=0)
    wc = W_c.reshape(1, _D)
    wt = W_t.reshape(1, _D)
    P = jnp.asarray(_P_np)
    OH = jnp.asarray(_OH_np)

    grid = (_B // _BB,)
    return pl.pallas_call(
        _embed_kernel,
        grid=grid,
        in_specs=[
            pl.BlockSpec((_BB, _NSP), lambda i: (i, 0)),
            pl.BlockSpec((_NSP, _L), lambda i: (0, 0)),
            pl.BlockSpec((_L, 8), lambda i: (0, 0)),
            pl.BlockSpec((8, _D), lambda i: (0, 0)),
            pl.BlockSpec((1, _D), lambda i: (0, 0)),
            pl.BlockSpec((1, _D), lambda i: (0, 0)),
        ],
        out_specs=pl.BlockSpec((_BB, _L, _D), lambda i: (i, 0, 0)),
        out_shape=jax.ShapeDtypeStruct((_B, _L, _D), jnp.float32),
    )(scal, P, OH, small, wc, wt)


# speedup vs baseline: 1.2820x; 1.0019x over previous
import numpy as np
import jax
import jax.numpy as jnp
from jax import lax
from jax.experimental import pallas as pl
from jax.experimental.pallas import tpu as pltpu

_B, _C, _R = 128, 24, 20
_NPORTS = 16
_M = _NPORTS * (_NPORTS - 1) // 2
_D = 256
_BAYL = _C * (_R + 1)
_L = _BAYL + _M + (_NPORTS - 2) + 1
_NS = _C * _R + _M
_NSP = 608


def _build_maps():
    src = np.full((_L,), -1, np.int64)
    biasrow = np.zeros((_L,), np.int64)
    for c in range(_C):
        for r in range(_R):
            p = c * (_R + 1) + r
            src[p] = c * _R + r
            biasrow[p] = 0
        p = c * (_R + 1) + _R
        biasrow[p] = 2 + 0
    biasrow[_BAYL - 1] = 2 + 1
    pos = _BAYL
    idx = 0
    for row_len in range(_NPORTS - 1, 0, -1):
        for _ in range(row_len):
            src[pos] = _C * _R + idx
            biasrow[pos] = 1
            idx += 1
            pos += 1
        if idx != _M:
            biasrow[pos] = 2 + 2
            pos += 1
    biasrow[pos] = 2 + 3
    pos += 1
    assert pos == _L
    return src, biasrow


_SRC, _BIASROW = _build_maps()

_P_np = np.zeros((_NSP, _L), np.float32)
for _p in range(_L):
    if _SRC[_p] >= 0:
        _P_np[_SRC[_p], _p] = 1.0
_OH_np = np.zeros((_L, 8), np.float32)
_OH_np[np.arange(_L), _BIASROW] = 1.0

_BB = 16


def _embed_kernel(scal_ref, p_ref, oh_ref, small_ref, wc_ref, wt_ref, out_ref):
    sg = jnp.dot(scal_ref[...], p_ref[...],
                 preferred_element_type=jnp.float32,
                 precision=lax.Precision.HIGHEST)
    bias = jnp.dot(oh_ref[...], small_ref[...],
                   preferred_element_type=jnp.float32,
                   precision=lax.Precision.HIGHEST)
    pos = lax.broadcasted_iota(jnp.int32, (_L, 1), 0)
    wsel = jnp.where(pos < _BAYL, wc_ref[...], wt_ref[...])
    out_ref[...] = sg[:, :, None] * wsel[None, :, :] + bias[None, :, :]


def kernel(bay, T, W_c, b_c, W_t, b_t, token_table):
    bay2 = bay.reshape(_B, _C * _R)
    T2 = T.reshape(_B, _M)
    scal = jnp.concatenate(
        [bay2, T2, jnp.zeros((_B, _NSP - _NS), jnp.float32)], axis=1)
    small = jnp.concatenate(
        [b_c[None, :], b_t[None, :], token_table,
         jnp.zeros((2, _D), jnp.float32)], axis=0)
    wc = W_c.reshape(1, _D)
    wt = W_t.reshape(1, _D)
    P = jnp.asarray(_P_np)
    OH = jnp.asarray(_OH_np)

    grid = (_B // _BB,)
    return pl.pallas_call(
        _embed_kernel,
        grid=grid,
        in_specs=[
            pl.BlockSpec((_BB, _NSP), lambda i: (i, 0)),
            pl.BlockSpec((_NSP, _L), lambda i: (0, 0)),
            pl.BlockSpec((_L, 8), lambda i: (0, 0)),
            pl.BlockSpec((8, _D), lambda i: (0, 0)),
            pl.BlockSpec((1, _D), lambda i: (0, 0)),
            pl.BlockSpec((1, _D), lambda i: (0, 0)),
        ],
        out_specs=pl.BlockSpec((_BB, _L, _D), lambda i: (i, 0, 0)),
        out_shape=jax.ShapeDtypeStruct((_B, _L, _D), jnp.float32),
        compiler_params=pltpu.CompilerParams(
            dimension_semantics=("parallel",)),
    )(scal, P, OH, small, wc, wt)
